# baseline (speedup 1.0000x reference)
import jax
import jax.numpy as jnp
from jax.experimental import pallas as pl
from jax.experimental.pallas import tpu as pltpu

_B, _N, _M = 8, 2048, 2048
_TI = 256
_NI = _N // _TI
_K = 16


def _bf16_split3(x):
    hi = x.astype(jnp.bfloat16)
    r1 = x - hi.astype(jnp.float32)
    mid = r1.astype(jnp.bfloat16)
    r2 = r1 - mid.astype(jnp.float32)
    lo = r2.astype(jnp.bfloat16)
    return hi, mid, lo


def _chamfer_tc_kernel(a_ref, bt_ref, out_ref, colmin_ref):
    b_i = pl.program_id(0)
    i = pl.program_id(1)

    a = a_ref[0]
    bt = bt_ref[0]

    ax, ay, az = a[:, 0:1], a[:, 1:2], a[:, 2:3]
    a2 = ax * ax + ay * ay + az * az
    a2h, a2m, a2l = _bf16_split3(a2)
    onesa = jnp.ones_like(a2, dtype=jnp.bfloat16)
    zerosa = jnp.zeros((_TI, _K - 9), dtype=jnp.bfloat16)
    a_aug = jnp.concatenate(
        [ax.astype(jnp.bfloat16), ay.astype(jnp.bfloat16),
         az.astype(jnp.bfloat16), a2h, a2m, a2l,
         onesa, onesa, onesa, zerosa], axis=1)

    bx, by, bz = bt[0:1, :], bt[1:2, :], bt[2:3, :]
    b2 = bx * bx + by * by + bz * bz
    b2h, b2m, b2l = _bf16_split3(b2)
    m2 = jnp.float32(-2.0)
    onesb = jnp.ones_like(b2, dtype=jnp.bfloat16)
    zerosb = jnp.zeros((_K - 9, _M), dtype=jnp.bfloat16)
    b_aug = jnp.concatenate(
        [(m2 * bx.astype(jnp.bfloat16).astype(jnp.float32)).astype(jnp.bfloat16),
         (m2 * by.astype(jnp.bfloat16).astype(jnp.float32)).astype(jnp.bfloat16),
         (m2 * bz.astype(jnp.bfloat16).astype(jnp.float32)).astype(jnp.bfloat16),
         onesb, onesb, onesb, b2h, b2m, b2l, zerosb], axis=0)

    d = jax.lax.dot_general(
        a_aug, b_aug, (((1,), (0,)), ((), ())),
        preferred_element_type=jnp.float32)

    rowmin = jnp.min(d, axis=1)
    colmin = jnp.min(d, axis=0, keepdims=True)

    @pl.when(jnp.logical_and(b_i == 0, i == 0))
    def _():
        out_ref[0] = 0.0

    @pl.when(i == 0)
    def _():
        colmin_ref[...] = colmin

    @pl.when(i != 0)
    def _():
        colmin_ref[...] = jnp.minimum(colmin_ref[...], colmin)

    out_ref[0] += jnp.sum(rowmin) * (1.0 / (_B * _N))

    @pl.when(i == _NI - 1)
    def _():
        out_ref[0] += jnp.sum(colmin_ref[...]) * (1.0 / (_B * _M))


def kernel(p1, p2):
    p2t = jnp.transpose(p2, (0, 2, 1))
    out = pl.pallas_call(
        _chamfer_tc_kernel,
        grid=(_B, _NI),
        in_specs=[
            pl.BlockSpec((1, _TI, 3), lambda b, i: (b, i, 0)),
            pl.BlockSpec((1, 3, _M), lambda b, i: (b, 0, 0)),
        ],
        out_specs=pl.BlockSpec(memory_space=pltpu.SMEM),
        out_shape=jax.ShapeDtypeStruct((1,), jnp.float32),
        scratch_shapes=[pltpu.VMEM((1, _M), jnp.float32)],
    )(p1, p2t)
    return out


# speedup vs baseline: 1.8958x; 1.8958x over previous
import jax
import jax.numpy as jnp
from jax.experimental import pallas as pl
from jax.experimental.pallas import tpu as pltpu

_B, _N, _M = 8, 2048, 2048
_TI = 1024
_NI = _N // _TI
_K = 16


def _bf16_split3(x):
    hi = x.astype(jnp.bfloat16)
    r1 = x - hi.astype(jnp.float32)
    mid = r1.astype(jnp.bfloat16)
    r2 = r1 - mid.astype(jnp.float32)
    lo = r2.astype(jnp.bfloat16)
    return hi, mid, lo


def _augment(p1, p2):
    a16 = p1.astype(jnp.bfloat16)
    a2 = jnp.sum(p1 * p1, axis=2, keepdims=True)
    a2h, a2m, a2l = _bf16_split3(a2)
    ones_a = jnp.ones_like(a2, dtype=jnp.bfloat16)
    zeros_a = jnp.zeros(a2.shape[:2] + (_K - 9,), dtype=jnp.bfloat16)
    a_aug = jnp.concatenate(
        [a16, a2h, a2m, a2l, ones_a, ones_a, ones_a, zeros_a], axis=2)

    b16m2 = (-2.0 * p2.astype(jnp.bfloat16).astype(jnp.float32)
             ).astype(jnp.bfloat16)
    b2 = jnp.sum(p2 * p2, axis=2, keepdims=True)
    b2h, b2m, b2l = _bf16_split3(b2)
    ones_b = jnp.ones_like(b2, dtype=jnp.bfloat16)
    zeros_b = jnp.zeros(b2.shape[:2] + (_K - 9,), dtype=jnp.bfloat16)
    b_aug = jnp.concatenate(
        [b16m2, ones_b, ones_b, ones_b, b2h, b2m, b2l, zeros_b], axis=2)
    return a_aug, jnp.transpose(b_aug, (0, 2, 1))


def _chamfer_tc_kernel(a_ref, b_ref, out_ref, colmin_ref):
    b_i = pl.program_id(0)
    i = pl.program_id(1)

    d = jax.lax.dot_general(
        a_ref[0], b_ref[0], (((1,), (0,)), ((), ())),
        preferred_element_type=jnp.float32)

    rowmin = jnp.min(d, axis=1)
    colmin = jnp.min(d, axis=0, keepdims=True)

    @pl.when(jnp.logical_and(b_i == 0, i == 0))
    def _():
        out_ref[0] = 0.0

    @pl.when(i == 0)
    def _():
        colmin_ref[...] = colmin

    @pl.when(i != 0)
    def _():
        colmin_ref[...] = jnp.minimum(colmin_ref[...], colmin)

    out_ref[0] += jnp.sum(rowmin) * (1.0 / (_B * _N))

    @pl.when(i == _NI - 1)
    def _():
        out_ref[0] += jnp.sum(colmin_ref[...]) * (1.0 / (_B * _M))


def kernel(p1, p2):
    a_aug, b_aug = _augment(p1, p2)
    out = pl.pallas_call(
        _chamfer_tc_kernel,
        grid=(_B, _NI),
        in_specs=[
            pl.BlockSpec((1, _TI, _K), lambda b, i: (b, i, 0)),
            pl.BlockSpec((1, _K, _M), lambda b, i: (b, 0, 0)),
        ],
        out_specs=pl.BlockSpec(memory_space=pltpu.SMEM),
        out_shape=jax.ShapeDtypeStruct((1,), jnp.float32),
        scratch_shapes=[pltpu.VMEM((1, _M), jnp.float32)],
    )(a_aug, b_aug)
    return out
